# initial kernel scaffold (unmeasured)
import jax
import jax.numpy as jnp
from jax import lax
from jax.experimental import pallas as pl
from jax.experimental.pallas import tpu as pltpu


def kernel(
    x,
):
    def body(*refs):
        pass

    out_shape = jax.ShapeDtypeStruct(..., jnp.float32)
    return pl.pallas_call(body, out_shape=out_shape)(...)


# baseline (device time: 32480 ns/iter reference)
import jax
import jax.numpy as jnp
from jax import lax
from jax.experimental import pallas as pl
from jax.experimental.pallas import tpu as pltpu

M = 1024
NCOL = 512


def kernel(x):
    def body(x_ref, out_ref, xbuf, rbuf_x, red_buf, rbuf_y, sems):
        my_x = lax.axis_index("x")
        my_y = lax.axis_index("y")
        x_nbr = (1 - my_x, my_y)
        y_nbr = (my_x, 1 - my_y)

        barrier = pltpu.get_barrier_semaphore()
        for nbr in (x_nbr, y_nbr):
            pl.semaphore_signal(
                barrier, inc=1, device_id=nbr,
                device_id_type=pl.DeviceIdType.MESH,
            )
        pl.semaphore_wait(barrier, 2)

        xbuf[...] = x_ref[0].astype(jnp.bfloat16)

        rdma1 = pltpu.make_async_remote_copy(
            src_ref=xbuf, dst_ref=rbuf_x,
            send_sem=sems.at[0], recv_sem=sems.at[1],
            device_id=x_nbr, device_id_type=pl.DeviceIdType.MESH,
        )
        rdma1.start()
        rdma1.wait()

        red_buf[...] = xbuf[...] + rbuf_x[...]
        out_ref[:, pl.ds(my_y * NCOL, NCOL)] = red_buf[...].astype(jnp.float32)

        rdma2 = pltpu.make_async_remote_copy(
            src_ref=red_buf, dst_ref=rbuf_y,
            send_sem=sems.at[2], recv_sem=sems.at[3],
            device_id=y_nbr, device_id_type=pl.DeviceIdType.MESH,
        )
        rdma2.start()
        rdma2.wait()

        out_ref[:, pl.ds((1 - my_y) * NCOL, NCOL)] = (
            rbuf_y[...].astype(jnp.float32)
        )

    return pl.pallas_call(
        body,
        out_shape=jax.ShapeDtypeStruct((M, 2 * NCOL), jnp.float32),
        in_specs=[pl.BlockSpec(memory_space=pltpu.VMEM)],
        out_specs=pl.BlockSpec(memory_space=pltpu.VMEM),
        scratch_shapes=[
            pltpu.VMEM((M, NCOL), jnp.bfloat16),
            pltpu.VMEM((M, NCOL), jnp.bfloat16),
            pltpu.VMEM((M, NCOL), jnp.bfloat16),
            pltpu.VMEM((M, NCOL), jnp.bfloat16),
            pltpu.SemaphoreType.DMA((4,)),
        ],
        compiler_params=pltpu.CompilerParams(collective_id=0),
    )(x)


# device time: 21799 ns/iter; 1.4900x vs baseline; 1.4900x over previous
import jax
import jax.numpy as jnp
from jax import lax
from jax.experimental import pallas as pl
from jax.experimental.pallas import tpu as pltpu

M = 1024
NCOL = 512
K = 8
RPC = M // K


def kernel(x):
    def body(x_ref, out_ref, xbuf, rbuf_x, red_buf, rbuf_y,
             send1, recv1, send2, recv2):
        my_x = lax.axis_index("x")
        my_y = lax.axis_index("y")
        x_nbr = (1 - my_x, my_y)
        y_nbr = (my_x, 1 - my_y)

        barrier = pltpu.get_barrier_semaphore()
        for nbr in (x_nbr, y_nbr):
            pl.semaphore_signal(
                barrier, inc=1, device_id=nbr,
                device_id_type=pl.DeviceIdType.MESH,
            )
        pl.semaphore_wait(barrier, 2)

        rdma1 = []
        for c in range(K):
            rows = pl.ds(c * RPC, RPC)
            xbuf[rows, :] = x_ref[0, rows, :].astype(jnp.bfloat16)
            r = pltpu.make_async_remote_copy(
                src_ref=xbuf.at[rows],
                dst_ref=rbuf_x.at[rows],
                send_sem=send1.at[c], recv_sem=recv1.at[c],
                device_id=x_nbr, device_id_type=pl.DeviceIdType.MESH,
            )
            r.start()
            rdma1.append(r)

        rdma2 = []
        for c in range(K):
            rows = pl.ds(c * RPC, RPC)
            rdma1[c].wait_recv()
            red_buf[rows, :] = xbuf[rows, :] + rbuf_x[rows, :]
            r = pltpu.make_async_remote_copy(
                src_ref=red_buf.at[rows],
                dst_ref=rbuf_y.at[rows],
                send_sem=send2.at[c], recv_sem=recv2.at[c],
                device_id=y_nbr, device_id_type=pl.DeviceIdType.MESH,
            )
            r.start()
            rdma2.append(r)
            out_ref[rows, pl.ds(my_y * NCOL, NCOL)] = red_buf[rows, :]

        for c in range(K):
            rows = pl.ds(c * RPC, RPC)
            rdma2[c].wait_recv()
            out_ref[rows, pl.ds((1 - my_y) * NCOL, NCOL)] = rbuf_y[rows, :]

        for c in range(K):
            rdma1[c].wait_send()
            rdma2[c].wait_send()

    return pl.pallas_call(
        body,
        out_shape=jax.ShapeDtypeStruct((M, 2 * NCOL), jnp.bfloat16),
        in_specs=[pl.BlockSpec(memory_space=pltpu.VMEM)],
        out_specs=pl.BlockSpec(memory_space=pltpu.VMEM),
        scratch_shapes=[
            pltpu.VMEM((M, NCOL), jnp.bfloat16),
            pltpu.VMEM((M, NCOL), jnp.bfloat16),
            pltpu.VMEM((M, NCOL), jnp.bfloat16),
            pltpu.VMEM((M, NCOL), jnp.bfloat16),
            pltpu.SemaphoreType.DMA((K,)),
            pltpu.SemaphoreType.DMA((K,)),
            pltpu.SemaphoreType.DMA((K,)),
            pltpu.SemaphoreType.DMA((K,)),
        ],
        compiler_params=pltpu.CompilerParams(collective_id=0),
    )(x)


# device time: 21484 ns/iter; 1.5118x vs baseline; 1.0147x over previous
import jax
import jax.numpy as jnp
from jax import lax
from jax.experimental import pallas as pl
from jax.experimental.pallas import tpu as pltpu

M = 1024
NCOL = 512
K = 16
RPC = M // K


def kernel(x):
    def body(x_ref, out_ref, xbuf, rbuf_x, red_buf, rbuf_y,
             send1, recv1, send2, recv2):
        my_x = lax.axis_index("x")
        my_y = lax.axis_index("y")
        x_nbr = (1 - my_x, my_y)
        y_nbr = (my_x, 1 - my_y)

        barrier = pltpu.get_barrier_semaphore()
        for nbr in (x_nbr, y_nbr):
            pl.semaphore_signal(
                barrier, inc=1, device_id=nbr,
                device_id_type=pl.DeviceIdType.MESH,
            )
        pl.semaphore_wait(barrier, 2)

        rdma1 = []
        for c in range(K):
            rows = pl.ds(c * RPC, RPC)
            xbuf[rows, :] = x_ref[0, rows, :].astype(jnp.bfloat16)
            r = pltpu.make_async_remote_copy(
                src_ref=xbuf.at[rows],
                dst_ref=rbuf_x.at[rows],
                send_sem=send1.at[c], recv_sem=recv1.at[c],
                device_id=x_nbr, device_id_type=pl.DeviceIdType.MESH,
            )
            r.start()
            rdma1.append(r)

        rdma2 = []
        for c in range(K):
            rows = pl.ds(c * RPC, RPC)
            rdma1[c].wait_recv()
            red_buf[rows, :] = xbuf[rows, :] + rbuf_x[rows, :]
            r = pltpu.make_async_remote_copy(
                src_ref=red_buf.at[rows],
                dst_ref=rbuf_y.at[rows],
                send_sem=send2.at[c], recv_sem=recv2.at[c],
                device_id=y_nbr, device_id_type=pl.DeviceIdType.MESH,
            )
            r.start()
            rdma2.append(r)
            @pl.when(my_y == 0)
            def _():
                out_ref[rows, :NCOL] = red_buf[rows, :]

            @pl.when(my_y == 1)
            def _():
                out_ref[rows, NCOL:] = red_buf[rows, :]

        for c in range(K):
            rows = pl.ds(c * RPC, RPC)
            rdma2[c].wait_recv()

            @pl.when(my_y == 0)
            def _():
                out_ref[rows, NCOL:] = rbuf_y[rows, :]

            @pl.when(my_y == 1)
            def _():
                out_ref[rows, :NCOL] = rbuf_y[rows, :]

        for c in range(K):
            rdma1[c].wait_send()
            rdma2[c].wait_send()

    return pl.pallas_call(
        body,
        out_shape=jax.ShapeDtypeStruct((M, 2 * NCOL), jnp.bfloat16),
        in_specs=[pl.BlockSpec(memory_space=pltpu.VMEM)],
        out_specs=pl.BlockSpec(memory_space=pltpu.VMEM),
        scratch_shapes=[
            pltpu.VMEM((M, NCOL), jnp.bfloat16),
            pltpu.VMEM((M, NCOL), jnp.bfloat16),
            pltpu.VMEM((M, NCOL), jnp.bfloat16),
            pltpu.VMEM((M, NCOL), jnp.bfloat16),
            pltpu.SemaphoreType.DMA((K,)),
            pltpu.SemaphoreType.DMA((K,)),
            pltpu.SemaphoreType.DMA((K,)),
            pltpu.SemaphoreType.DMA((K,)),
        ],
        compiler_params=pltpu.CompilerParams(collective_id=0),
    )(x)
